# Initial kernel scaffold; baseline (speedup 1.0000x reference)
import jax
import jax.numpy as jnp
from jax.experimental import pallas as pl


def kernel(x):
    raise NotImplementedError("write your pallas kernel here")


# speedup vs baseline: 1.0298x; 1.0298x over previous
import functools

import jax
import jax.numpy as jnp
from jax import lax
from jax.experimental import pallas as pl
from jax.experimental.pallas import tpu as pltpu
from jax.experimental.pallas import tpu_sc as plsc

R, C = 128, 32768
NC, NS, L = 2, 16, 16
NW = NC * NS
RPW = R // NW
CH = 16384
NCH_ROW = C // CH
NTASK = RPW * NCH_ROW
UNROLL = 8
NACC = 4

_mesh = plsc.VectorSubcoreMesh(core_axis_name="c", subcore_axis_name="s")


@functools.partial(
    pl.kernel,
    out_type=jax.ShapeDtypeStruct((NW, L), jnp.float32),
    mesh=_mesh,
    scratch_types=[
        pltpu.VMEM((CH,), jnp.float32),
        pltpu.VMEM((CH,), jnp.float32),
        pltpu.VMEM((L,), jnp.float32),
        pltpu.SemaphoreType.DMA,
        pltpu.SemaphoreType.DMA,
    ],
)
def _sc_rowmax(x_hbm, out_hbm, buf0, buf1, res_v, sem0, sem1):
    wid = lax.axis_index("s") * NC + lax.axis_index("c")
    base_row = wid * RPW
    bufs = (buf0, buf1)
    sems = (sem0, sem1)

    def start(t):
        r = t // NCH_ROW
        c = t % NCH_ROW
        cp = pltpu.make_async_copy(
            x_hbm.at[base_row + r, pl.ds(c * CH, CH)], bufs[t % 2], sems[t % 2]
        )
        cp.start()
        return cp

    copies = [start(0), None]

    lane = lax.iota(jnp.int32, L)
    res = jnp.zeros((L,), jnp.float32)
    accs = [jnp.zeros((L,), jnp.float32) for _ in range(NACC)]

    for t in range(NTASK):
        if t + 1 < NTASK:
            copies[(t + 1) % 2] = start(t + 1)
        copies[t % 2].wait()
        buf = bufs[t % 2]

        def body(i, a, buf=buf):
            a = list(a)
            for u in range(UNROLL):
                v = buf[pl.ds((i * UNROLL + u) * L, L)]
                a[u % NACC] = jnp.maximum(a[u % NACC], v * v)
            return tuple(a)

        accs = list(lax.fori_loop(0, CH // (L * UNROLL), body, tuple(accs)))

        if (t + 1) % NCH_ROW == 0:
            m01 = jnp.maximum(accs[0], accs[1])
            m23 = jnp.maximum(accs[2], accs[3])
            merged = jnp.maximum(m01, m23)
            vals = [merged[i] for i in range(L)]
            while len(vals) > 1:
                vals = [
                    jnp.maximum(vals[2 * i], vals[2 * i + 1])
                    for i in range(len(vals) // 2)
                ]
            r = t // NCH_ROW
            res = jnp.where(lane == r, vals[0], res)
            accs = [jnp.zeros((L,), jnp.float32) for _ in range(NACC)]

    res_v[...] = res
    pltpu.sync_copy(res_v, out_hbm.at[wid])


def kernel(x):
    out2d = _sc_rowmax(x)
    return out2d[:, :RPW].reshape(R)


# speedup vs baseline: 1.0453x; 1.0150x over previous
import functools

import jax
import jax.numpy as jnp
from jax import lax
from jax.experimental import pallas as pl
from jax.experimental.pallas import tpu as pltpu
from jax.experimental.pallas import tpu_sc as plsc

R, C = 128, 32768
NC, NS, L = 2, 16, 16
NW = NC * NS
RPW = R // NW
CH = 32768
NCH_ROW = C // CH
NTASK = RPW * NCH_ROW
UNROLL = 8
NACC = 4

_mesh = plsc.VectorSubcoreMesh(core_axis_name="c", subcore_axis_name="s")


@functools.partial(
    pl.kernel,
    out_type=jax.ShapeDtypeStruct((NW, L), jnp.float32),
    mesh=_mesh,
    scratch_types=[
        pltpu.VMEM((CH,), jnp.float32),
        pltpu.VMEM((CH,), jnp.float32),
        pltpu.VMEM((L,), jnp.float32),
        pltpu.SemaphoreType.DMA,
        pltpu.SemaphoreType.DMA,
    ],
)
def _sc_rowmax(x_hbm, out_hbm, buf0, buf1, res_v, sem0, sem1):
    wid = lax.axis_index("s") * NC + lax.axis_index("c")
    base_row = wid * RPW
    bufs = (buf0, buf1)
    sems = (sem0, sem1)

    def start(t):
        r = t // NCH_ROW
        c = t % NCH_ROW
        cp = pltpu.make_async_copy(
            x_hbm.at[base_row + r, pl.ds(c * CH, CH)], bufs[t % 2], sems[t % 2]
        )
        cp.start()
        return cp

    copies = [start(0), None]

    lane = lax.iota(jnp.int32, L)
    res = jnp.zeros((L,), jnp.float32)
    accs = [jnp.zeros((L,), jnp.float32) for _ in range(NACC)]

    for t in range(NTASK):
        if t + 1 < NTASK:
            copies[(t + 1) % 2] = start(t + 1)
        copies[t % 2].wait()
        buf = bufs[t % 2]

        def body(i, a, buf=buf):
            a = list(a)
            for u in range(UNROLL):
                v = buf[pl.ds((i * UNROLL + u) * L, L)]
                a[u % NACC] = jnp.maximum(a[u % NACC], v * v)
            return tuple(a)

        accs = list(lax.fori_loop(0, CH // (L * UNROLL), body, tuple(accs)))

        if (t + 1) % NCH_ROW == 0:
            m01 = jnp.maximum(accs[0], accs[1])
            m23 = jnp.maximum(accs[2], accs[3])
            merged = jnp.maximum(m01, m23)
            vals = [merged[i] for i in range(L)]
            while len(vals) > 1:
                vals = [
                    jnp.maximum(vals[2 * i], vals[2 * i + 1])
                    for i in range(len(vals) // 2)
                ]
            r = t // NCH_ROW
            res = jnp.where(lane == r, vals[0], res)
            accs = [jnp.zeros((L,), jnp.float32) for _ in range(NACC)]

    res_v[...] = res
    pltpu.sync_copy(res_v, out_hbm.at[wid])


def kernel(x):
    out2d = _sc_rowmax(x)
    return out2d[:, :RPW].reshape(R)
